# baseline (speedup 1.0000x reference)
import functools

import jax
import jax.numpy as jnp
from jax import lax
from jax.experimental import pallas as pl
from jax.experimental.pallas import tpu as pltpu
from jax.experimental.pallas import tpu_sc as plsc

_NUM_CORES = 2
_NUM_SUBCORES = 16
_NUM_WORKERS = _NUM_CORES * _NUM_SUBCORES

_D = 512
_CHUNK = 32
_NBUF = 6


@functools.partial(jax.jit, static_argnames=("b_per_w", "n_chunks"))
def _sc_gather(idx, table, *, b_per_w, n_chunks):
    mesh = plsc.VectorSubcoreMesh(
        core_axis_name="c", subcore_axis_name="s",
        num_cores=_NUM_CORES, num_subcores=_NUM_SUBCORES,
    )
    B = b_per_w * _NUM_WORKERS

    @functools.partial(
        pl.kernel,
        out_type=jax.ShapeDtypeStruct((B, _D), jnp.float32),
        mesh=mesh,
        scratch_types=[
            pltpu.VMEM((n_chunks, _CHUNK), jnp.int32),
            pltpu.VMEM((_NBUF, _CHUNK, _D), jnp.float32),
            pltpu.SemaphoreType.DMA,
            pltpu.SemaphoreType.DMA,
        ]
        + [pltpu.SemaphoreType.DMA] * (2 * _NBUF),
    )
    def k(idx_hbm, table_hbm, out_hbm, idx_v, rows_v, isem, isem2, *sems):
        gsem = sems[:_NBUF]
        osem = sems[_NBUF:]
        wid = lax.axis_index("s") * _NUM_CORES + lax.axis_index("c")
        base = wid * b_per_w
        irow = wid * n_chunks
        head = min(8, n_chunks)
        ic0 = pltpu.async_copy(
            idx_hbm.at[pl.ds(irow, head)], idx_v.at[pl.ds(0, head)], isem)
        ic_rest = None
        if n_chunks > head:
            ic_rest = pltpu.async_copy(
                idx_hbm.at[pl.ds(irow + head, n_chunks - head)],
                idx_v.at[pl.ds(head, n_chunks - head)], isem2)

        gather = [None] * _NBUF
        out_cp = [None] * _NBUF

        def start_gather(c, b):
            return pltpu.async_copy(
                table_hbm.at[idx_v.at[c]], rows_v.at[b], gsem[b])

        ic0.wait()
        for j in range(min(_NBUF - 1, n_chunks)):
            gather[j] = start_gather(j, j)
        if ic_rest is not None:
            ic_rest.wait()

        for i in range(n_chunks):
            b = i % _NBUF
            pf = i + _NBUF - 1
            if pf < n_chunks:
                pb = pf % _NBUF
                if out_cp[pb] is not None:
                    out_cp[pb].wait()
                gather[pb] = start_gather(pf, pb)
            gather[b].wait()
            out_cp[b] = pltpu.async_copy(
                rows_v.at[b], out_hbm.at[pl.ds(base + i * _CHUNK, _CHUNK)],
                osem[b])
        for b in range(_NBUF):
            if out_cp[b] is not None:
                out_cp[b].wait()

    return k(idx, table)


def kernel(x, emb_weight):
    B = x.shape[0] * x.shape[1]
    idx = x.reshape((B,)).astype(jnp.int32)
    b_per_w = B // _NUM_WORKERS
    n_chunks = b_per_w // _CHUNK
    idx2d = idx.reshape((_NUM_WORKERS * n_chunks, _CHUNK))
    out = _sc_gather(idx2d, emb_weight, b_per_w=b_per_w, n_chunks=n_chunks)
    return out.reshape((x.shape[0], x.shape[1], _D))


# speedup vs baseline: 1.0111x; 1.0111x over previous
import functools

import jax
import jax.numpy as jnp
from jax import lax
from jax.experimental import pallas as pl
from jax.experimental.pallas import tpu as pltpu
from jax.experimental.pallas import tpu_sc as plsc

_NUM_CORES = 2
_NUM_SUBCORES = 16
_NUM_WORKERS = _NUM_CORES * _NUM_SUBCORES

_D = 512
_CHUNK = 32
_NBUF = 6


@functools.partial(jax.jit, static_argnames=("b_per_w", "n_chunks"))
def _sc_gather(idx, table, *, b_per_w, n_chunks):
    mesh = plsc.VectorSubcoreMesh(
        core_axis_name="c", subcore_axis_name="s",
        num_cores=_NUM_CORES, num_subcores=_NUM_SUBCORES,
    )
    B = b_per_w * _NUM_WORKERS

    @functools.partial(
        pl.kernel,
        out_type=jax.ShapeDtypeStruct((B, _D), jnp.float32),
        mesh=mesh,
        scratch_types=[
            pltpu.VMEM((b_per_w,), jnp.int32),
            pltpu.VMEM((_NBUF, _CHUNK, _D), jnp.float32),
            pltpu.SemaphoreType.DMA,
            pltpu.SemaphoreType.DMA,
        ]
        + [pltpu.SemaphoreType.DMA] * (2 * _NBUF),
    )
    def k(idx_hbm, table_hbm, out_hbm, idx_v, rows_v, isem, isem2, *sems):
        gsem = sems[:_NBUF]
        osem = sems[_NBUF:]
        wid = lax.axis_index("s") * _NUM_CORES + lax.axis_index("c")
        base = wid * b_per_w
        head = min(8 * _CHUNK, b_per_w)
        ic0 = pltpu.async_copy(
            idx_hbm.at[pl.ds(base, head)], idx_v.at[pl.ds(0, head)], isem)
        ic_rest = None
        if b_per_w > head:
            ic_rest = pltpu.async_copy(
                idx_hbm.at[pl.ds(base + head, b_per_w - head)],
                idx_v.at[pl.ds(head, b_per_w - head)], isem2)

        gather = [None] * _NBUF
        out_cp = [None] * _NBUF

        def start_gather(c, b):
            return pltpu.async_copy(
                table_hbm.at[idx_v.at[pl.ds(c * _CHUNK, _CHUNK)]],
                rows_v.at[b], gsem[b])

        ic0.wait()
        for j in range(min(_NBUF - 1, n_chunks)):
            gather[j] = start_gather(j, j)
        if ic_rest is not None:
            ic_rest.wait()

        for i in range(n_chunks):
            b = i % _NBUF
            pf = i + _NBUF - 1
            if pf < n_chunks:
                pb = pf % _NBUF
                if out_cp[pb] is not None:
                    out_cp[pb].wait()
                gather[pb] = start_gather(pf, pb)
            gather[b].wait()
            out_cp[b] = pltpu.async_copy(
                rows_v.at[b], out_hbm.at[pl.ds(base + i * _CHUNK, _CHUNK)],
                osem[b])
        for b in range(_NBUF):
            if out_cp[b] is not None:
                out_cp[b].wait()

    return k(idx, table)


def kernel(x, emb_weight):
    B = x.shape[0] * x.shape[1]
    idx = x.reshape((B,)).astype(jnp.int32)
    b_per_w = B // _NUM_WORKERS
    n_chunks = b_per_w // _CHUNK
    out = _sc_gather(idx, emb_weight, b_per_w=b_per_w, n_chunks=n_chunks)
    return out.reshape((x.shape[0], x.shape[1], _D))
